# baseline (speedup 1.0000x reference)
import functools

import jax
import jax.numpy as jnp
from jax import lax
from jax.experimental import pallas as pl
from jax.experimental.pallas import tpu as pltpu
from jax.experimental.pallas import tpu_sc as plsc

HID = 128
N_ATOM = 10000
N_QUERY = 50000
C_EDGE = 128
EBLK = 16 * C_EDGE


def _pad_to(n, m):
    return ((n + m - 1) // m) * m


def _affine_silu_body(x_ref, w_ref, b_ref, o_ref):
    o_ref[...] = jax.nn.silu(
        jnp.dot(x_ref[...], w_ref[...], preferred_element_type=jnp.float32)
        + b_ref[...])


def _affine_silu(x, w, b, block=1024):
    n, k = x.shape
    m = w.shape[1]
    return pl.pallas_call(
        _affine_silu_body,
        grid=(pl.cdiv(n, block),),
        in_specs=[
            pl.BlockSpec((block, k), lambda i: (i, 0)),
            pl.BlockSpec((k, m), lambda i: (0, 0)),
            pl.BlockSpec((1, m), lambda i: (0, 0)),
        ],
        out_specs=pl.BlockSpec((block, m), lambda i: (i, 0)),
        out_shape=jax.ShapeDtypeStruct((n, m), jnp.float32),
    )(x, w, b.reshape(1, -1))


def _update_body(h_ref, agg_ref, wa_ref, wb_ref, b1_ref, w2_ref, b2_ref, o_ref):
    h = h_ref[...]
    u = jax.nn.silu(
        jnp.dot(h, wa_ref[...], preferred_element_type=jnp.float32)
        + jnp.dot(agg_ref[...], wb_ref[...], preferred_element_type=jnp.float32)
        + b1_ref[...])
    o_ref[...] = h + jnp.dot(u, w2_ref[...], preferred_element_type=jnp.float32) + b2_ref[...]


def _update(h, agg, p, block=1024):
    n = h.shape[0]
    return pl.pallas_call(
        _update_body,
        grid=(pl.cdiv(n, block),),
        in_specs=[
            pl.BlockSpec((block, HID), lambda i: (i, 0)),
            pl.BlockSpec((block, HID), lambda i: (i, 0)),
            pl.BlockSpec((HID, HID), lambda i: (0, 0)),
            pl.BlockSpec((HID, HID), lambda i: (0, 0)),
            pl.BlockSpec((1, HID), lambda i: (0, 0)),
            pl.BlockSpec((HID, HID), lambda i: (0, 0)),
            pl.BlockSpec((1, HID), lambda i: (0, 0)),
        ],
        out_specs=pl.BlockSpec((block, HID), lambda i: (i, 0)),
        out_shape=jax.ShapeDtypeStruct((n, HID), jnp.float32),
    )(h, agg, p['Wu1'][:HID], p['Wu1'][HID:],
      p['bu1'].reshape(1, -1), p['Wu2'], p['bu2'].reshape(1, -1))


def _head_body(h_ref, w1_ref, b1_ref, w2_ref, b2_ref, o_ref):
    t = jax.nn.silu(
        jnp.dot(h_ref[...], w1_ref[...], preferred_element_type=jnp.float32)
        + b1_ref[...])
    o_ref[...] = jnp.dot(t, w2_ref[...], preferred_element_type=jnp.float32) + b2_ref[...]


def _head(h, params, block=1024):
    n = h.shape[0]
    hh = HID // 2
    return pl.pallas_call(
        _head_body,
        grid=(pl.cdiv(n, block),),
        in_specs=[
            pl.BlockSpec((block, HID), lambda i: (i, 0)),
            pl.BlockSpec((HID, hh), lambda i: (0, 0)),
            pl.BlockSpec((1, hh), lambda i: (0, 0)),
            pl.BlockSpec((hh, 1), lambda i: (0, 0)),
            pl.BlockSpec((1, 1), lambda i: (0, 0)),
        ],
        out_specs=pl.BlockSpec((block, 1), lambda i: (i, 0)),
        out_shape=jax.ShapeDtypeStruct((n, 1), jnp.float32),
    )(h, params['Wh1'], params['bh1'].reshape(1, -1),
      params['Wh2'], params['bh2'].reshape(1, -1))


def _edge_sc(hm, e, src_r, dst_r, zeros_hbm, n_pad, nblk):
    R = n_pad // 4
    acc_rows = R + 128
    rpt_z = acc_rows // 16
    rpt_f = R // 16
    rows_pt = nblk
    mesh = plsc.VectorSubcoreMesh(core_axis_name="c", subcore_axis_name="s")

    @functools.partial(
        pl.kernel, mesh=mesh,
        out_type=jax.ShapeDtypeStruct((n_pad, HID), jnp.float32),
        scratch_types=[
            pltpu.VMEM_SHARED((acc_rows, HID), jnp.float32),
            pltpu.VMEM((1, 128), jnp.int32),
            pltpu.VMEM((1, 128), jnp.int32),
            pltpu.VMEM((1, 128), jnp.int32),
            pltpu.VMEM((C_EDGE, HID), jnp.float32),
            pltpu.VMEM((64, HID), jnp.float32),
            pltpu.SemaphoreType.DMA,
        ],
    )
    def body(hm_h, e_h, src_h, dst_h, z_h, agg_h,
             acc, src_v, dst_v, sidx_v, rows_v, e_v, sem):
        c = lax.axis_index("c")
        s = lax.axis_index("s")
        for p in range(2):
            chunk = 2 * p + c
            lo = chunk * R
            off = 0
            while off < rpt_z:
                step = min(128, rpt_z - off)
                pltpu.sync_copy(z_h.at[pl.ds(0, step)],
                                acc.at[pl.ds(s * rpt_z + off, step)])
                off += step
            plsc.subcore_barrier()

            def blk(b, carry):
                base = s * rows_pt + b
                pltpu.sync_copy(src_h.at[pl.ds(base, 1)], src_v)
                pltpu.sync_copy(dst_h.at[pl.ds(base, 1)], dst_v)
                pltpu.async_copy(hm_h.at[src_v.at[0]], rows_v, sem).wait()
                for k in range(8):
                    d = dst_v[0, pl.ds(k * 16, 16)]
                    local = d - lo
                    ok = (local >= 0) & (local < R)
                    sidx_v[0, pl.ds(k * 16, 16)] = jnp.where(ok, local, R)
                for h in range(2):
                    pltpu.sync_copy(e_h.at[pl.ds(base * 128 + h * 64, 64)], e_v)

                    def mul(r, carry2):
                        for k in range(8):
                            rows_v[h * 64 + r, pl.ds(k * 16, 16)] = (
                                rows_v[h * 64 + r, pl.ds(k * 16, 16)]
                                * e_v[r, pl.ds(k * 16, 16)])
                        return carry2
                    lax.fori_loop(0, 64, mul, 0)
                pltpu.sync_copy(rows_v, acc.at[sidx_v.at[0]], add=True)
                return carry
            lax.fori_loop(0, nblk, blk, 0)
            plsc.subcore_barrier()
            off = 0
            while off < rpt_f:
                step = min(128, rpt_f - off)
                pltpu.sync_copy(acc.at[pl.ds(s * rpt_f + off, step)],
                                agg_h.at[pl.ds(lo + s * rpt_f + off, step)])
                off += step
            plsc.subcore_barrier()

    return body(hm, e, src_r, dst_r, zeros_hbm)


def _prep_edges(edge_index, n_dst):
    src = edge_index[0]
    dst = edge_index[1]
    e_num = src.shape[0]
    e_pad = _pad_to(e_num, EBLK)
    src_r = jnp.pad(src, (0, e_pad - e_num)).astype(jnp.int32).reshape(e_pad // 128, 128)
    dst_r = jnp.pad(dst, (0, e_pad - e_num),
                    constant_values=n_dst).astype(jnp.int32).reshape(e_pad // 128, 128)
    return src_r, dst_r, e_pad


def _round_sc(p, h_src, h_dst, src_r, dst_r, e, zeros_hbm, n_pad, nblk):
    hm = _affine_silu(h_src, p['Wm'], p['bm'])
    agg = _edge_sc(hm, e, src_r, dst_r, zeros_hbm, n_pad, nblk)
    return _update(h_dst, agg, p)


def _round_jnp(p, h_src, h_dst, src, dst, e, n_dst):
    hm = jax.nn.silu(h_src @ p['Wm'] + p['bm'])
    m = jnp.take(hm, src, axis=0) * e
    agg = jax.ops.segment_sum(m, dst, num_segments=n_dst)
    u = jnp.concatenate([h_dst, agg], axis=-1)
    return h_dst + jax.nn.silu(u @ p['Wu1'] + p['bu1']) @ p['Wu2'] + p['bu2']


def kernel(z, bond_edge_index, bond_edge_attr, aq_edge_index, aq_edge_attr,
           qq_edge_index, qq_edge_attr, n_query, params):
    p = params
    zeros_hbm = jnp.zeros((128, HID), jnp.float32)
    h_atom = jnp.take(p['emb'], z, axis=0)

    n_pad_b = _pad_to(N_ATOM + 1, 1024)
    src_b, dst_b, e_pad_b = _prep_edges(bond_edge_index, N_ATOM)
    attr_b = jnp.pad(bond_edge_attr, ((0, e_pad_b - bond_edge_attr.shape[0]), (0, 0)))
    e_b = _affine_silu(attr_b, p['bond']['Wr'], p['bond']['br'])
    nblk_b = e_pad_b // EBLK
    for _ in range(2):
        h_atom = _round_sc(p['bond'], h_atom, h_atom, src_b, dst_b, e_b,
                           zeros_hbm, n_pad_b, nblk_b)

    n_pad_q = _pad_to(N_QUERY + 1, 1024)
    src_a, dst_a, e_pad_a = _prep_edges(aq_edge_index, N_QUERY)
    attr_a = jnp.pad(aq_edge_attr, ((0, e_pad_a - aq_edge_attr.shape[0]), (0, 0)))
    e_a = _affine_silu(attr_a, p['aq']['Wr'], p['aq']['br'])
    nblk_a = e_pad_a // EBLK
    h_query = jnp.zeros((N_QUERY, HID), jnp.float32)
    for _ in range(3):
        h_query = _round_sc(p['aq'], h_atom, h_query, src_a, dst_a, e_a,
                            zeros_hbm, n_pad_q, nblk_a)

    src_q, dst_q, e_pad_q = _prep_edges(qq_edge_index, N_QUERY)
    attr_q = jnp.pad(qq_edge_attr, ((0, e_pad_q - qq_edge_attr.shape[0]), (0, 0)))
    e_q = _affine_silu(attr_q, p['qq']['Wr'], p['qq']['br'])
    nblk_q = e_pad_q // EBLK
    for _ in range(2):
        h_query = _round_sc(p['qq'], h_query, h_query, src_q, dst_q, e_q,
                            zeros_hbm, n_pad_q, nblk_q)

    return _head(h_query, p).reshape(N_QUERY)


# speedup vs baseline: 1.1162x; 1.1162x over previous
import functools

import jax
import jax.numpy as jnp
from jax import lax
from jax.experimental import pallas as pl
from jax.experimental.pallas import tpu as pltpu
from jax.experimental.pallas import tpu_sc as plsc

HID = 128
N_ATOM = 10000
N_QUERY = 50000
C_EDGE = 128
EBLK = 16 * C_EDGE


def _pad_to(n, m):
    return ((n + m - 1) // m) * m


def _affine_silu_body(x_ref, w_ref, b_ref, o_ref):
    o_ref[...] = jax.nn.silu(
        jnp.dot(x_ref[...], w_ref[...], preferred_element_type=jnp.float32)
        + b_ref[...])


def _affine_silu(x, w, b, block=1024):
    n, k = x.shape
    m = w.shape[1]
    return pl.pallas_call(
        _affine_silu_body,
        grid=(pl.cdiv(n, block),),
        in_specs=[
            pl.BlockSpec((block, k), lambda i: (i, 0)),
            pl.BlockSpec((k, m), lambda i: (0, 0)),
            pl.BlockSpec((1, m), lambda i: (0, 0)),
        ],
        out_specs=pl.BlockSpec((block, m), lambda i: (i, 0)),
        out_shape=jax.ShapeDtypeStruct((n, m), jnp.float32),
    )(x, w, b.reshape(1, -1))


def _update_body(h_ref, agg_ref, wa_ref, wb_ref, b1_ref, w2_ref, b2_ref, o_ref):
    h = h_ref[...]
    u = jax.nn.silu(
        jnp.dot(h, wa_ref[...], preferred_element_type=jnp.float32)
        + jnp.dot(agg_ref[...], wb_ref[...], preferred_element_type=jnp.float32)
        + b1_ref[...])
    o_ref[...] = h + jnp.dot(u, w2_ref[...], preferred_element_type=jnp.float32) + b2_ref[...]


def _update(h, agg, p, block=1024):
    n = h.shape[0]
    return pl.pallas_call(
        _update_body,
        grid=(pl.cdiv(n, block),),
        in_specs=[
            pl.BlockSpec((block, HID), lambda i: (i, 0)),
            pl.BlockSpec((block, HID), lambda i: (i, 0)),
            pl.BlockSpec((HID, HID), lambda i: (0, 0)),
            pl.BlockSpec((HID, HID), lambda i: (0, 0)),
            pl.BlockSpec((1, HID), lambda i: (0, 0)),
            pl.BlockSpec((HID, HID), lambda i: (0, 0)),
            pl.BlockSpec((1, HID), lambda i: (0, 0)),
        ],
        out_specs=pl.BlockSpec((block, HID), lambda i: (i, 0)),
        out_shape=jax.ShapeDtypeStruct((n, HID), jnp.float32),
    )(h, agg, p['Wu1'][:HID], p['Wu1'][HID:],
      p['bu1'].reshape(1, -1), p['Wu2'], p['bu2'].reshape(1, -1))


def _head_body(h_ref, w1_ref, b1_ref, w2_ref, b2_ref, o_ref):
    t = jax.nn.silu(
        jnp.dot(h_ref[...], w1_ref[...], preferred_element_type=jnp.float32)
        + b1_ref[...])
    o_ref[...] = jnp.dot(t, w2_ref[...], preferred_element_type=jnp.float32) + b2_ref[...]


def _head(h, params, block=1024):
    n = h.shape[0]
    hh = HID // 2
    return pl.pallas_call(
        _head_body,
        grid=(pl.cdiv(n, block),),
        in_specs=[
            pl.BlockSpec((block, HID), lambda i: (i, 0)),
            pl.BlockSpec((HID, hh), lambda i: (0, 0)),
            pl.BlockSpec((1, hh), lambda i: (0, 0)),
            pl.BlockSpec((hh, 1), lambda i: (0, 0)),
            pl.BlockSpec((1, 1), lambda i: (0, 0)),
        ],
        out_specs=pl.BlockSpec((block, 1), lambda i: (i, 0)),
        out_shape=jax.ShapeDtypeStruct((n, 1), jnp.float32),
    )(h, params['Wh1'], params['bh1'].reshape(1, -1),
      params['Wh2'], params['bh2'].reshape(1, -1))


def _edge_sc(hm, e, src_r, dst_r, zeros_hbm, n_pad, nblk):
    R = n_pad // 4
    acc_rows = R + 128
    rpt_z = acc_rows // 16
    rpt_f = R // 16
    rows_pt = nblk
    mesh = plsc.VectorSubcoreMesh(core_axis_name="c", subcore_axis_name="s")

    @functools.partial(
        pl.kernel, mesh=mesh,
        out_type=jax.ShapeDtypeStruct((n_pad, HID), jnp.float32),
        scratch_types=[
            pltpu.VMEM_SHARED((acc_rows, HID), jnp.float32),
            pltpu.VMEM((1, 128), jnp.int32),
            pltpu.VMEM((1, 128), jnp.int32),
            pltpu.VMEM((1, 128), jnp.int32),
            pltpu.VMEM((C_EDGE, HID), jnp.float32),
            pltpu.VMEM((64, HID), jnp.float32),
            pltpu.SemaphoreType.DMA,
        ],
    )
    def body(hm_h, e_h, src_h, dst_h, z_h, agg_h,
             acc, src_v, dst_v, sidx_v, rows_v, e_v, sem):
        c = lax.axis_index("c")
        s = lax.axis_index("s")
        for p in range(2):
            chunk = 2 * p + c
            lo = chunk * R
            off = 0
            while off < rpt_z:
                step = min(128, rpt_z - off)
                pltpu.sync_copy(z_h.at[pl.ds(0, step)],
                                acc.at[pl.ds(s * rpt_z + off, step)])
                off += step
            plsc.subcore_barrier()

            def blk(b, carry):
                base = s * rows_pt + b
                pltpu.sync_copy(src_h.at[pl.ds(base, 1)], src_v)
                pltpu.sync_copy(dst_h.at[pl.ds(base, 1)], dst_v)
                pltpu.async_copy(hm_h.at[src_v.at[0]], rows_v, sem).wait()
                for k in range(8):
                    d = dst_v[0, pl.ds(k * 16, 16)]
                    local = d - lo
                    ok = (local >= 0) & (local < R)
                    sidx_v[0, pl.ds(k * 16, 16)] = jnp.where(ok, local, R)
                for h in range(2):
                    pltpu.sync_copy(e_h.at[pl.ds(base * 128 + h * 64, 64)], e_v)

                    def mul(r, carry2):
                        for k in range(8):
                            rows_v[h * 64 + r, pl.ds(k * 16, 16)] = (
                                rows_v[h * 64 + r, pl.ds(k * 16, 16)]
                                * e_v[r, pl.ds(k * 16, 16)])
                        return carry2
                    lax.fori_loop(0, 64, mul, 0)
                pltpu.sync_copy(rows_v, acc.at[sidx_v.at[0]], add=True)
                return carry
            lax.fori_loop(0, nblk, blk, 0)
            plsc.subcore_barrier()
            off = 0
            while off < rpt_f:
                step = min(128, rpt_f - off)
                pltpu.sync_copy(acc.at[pl.ds(s * rpt_f + off, step)],
                                agg_h.at[pl.ds(lo + s * rpt_f + off, step)])
                off += step
            plsc.subcore_barrier()

    return body(hm, e, src_r, dst_r, zeros_hbm)


def _edge_sc_partial(hm, e, src_r, dst_r, zeros_hbm, n_pad, nblk32):
    rpt = n_pad // 16
    mesh = plsc.VectorSubcoreMesh(core_axis_name="c", subcore_axis_name="s")

    @functools.partial(
        pl.kernel, mesh=mesh,
        out_type=jax.ShapeDtypeStruct((2, n_pad, HID), jnp.float32),
        scratch_types=[
            pltpu.VMEM_SHARED((n_pad, HID), jnp.float32),
            pltpu.VMEM((1, 128), jnp.int32),
            pltpu.VMEM((1, 128), jnp.int32),
            pltpu.VMEM((C_EDGE, HID), jnp.float32),
            pltpu.VMEM((64, HID), jnp.float32),
            pltpu.SemaphoreType.DMA,
        ],
    )
    def body(hm_h, e_h, src_h, dst_h, z_h, agg_h,
             acc, src_v, dst_v, rows_v, e_v, sem):
        c = lax.axis_index("c")
        s = lax.axis_index("s")
        off = 0
        while off < rpt:
            step = min(128, rpt - off)
            pltpu.sync_copy(z_h.at[pl.ds(0, step)],
                            acc.at[pl.ds(s * rpt + off, step)])
            off += step
        plsc.subcore_barrier()

        def blk(b, carry):
            base = (c * 16 + s) * nblk32 + b
            pltpu.sync_copy(src_h.at[pl.ds(base, 1)], src_v)
            pltpu.sync_copy(dst_h.at[pl.ds(base, 1)], dst_v)
            pltpu.async_copy(hm_h.at[src_v.at[0]], rows_v, sem).wait()
            for h in range(2):
                pltpu.sync_copy(e_h.at[pl.ds(base * 128 + h * 64, 64)], e_v)

                def mul(r, carry2):
                    for k in range(8):
                        rows_v[h * 64 + r, pl.ds(k * 16, 16)] = (
                            rows_v[h * 64 + r, pl.ds(k * 16, 16)]
                            * e_v[r, pl.ds(k * 16, 16)])
                    return carry2
                lax.fori_loop(0, 64, mul, 0)
            pltpu.sync_copy(rows_v, acc.at[dst_v.at[0]], add=True)
            return carry
        lax.fori_loop(0, nblk32, blk, 0)
        plsc.subcore_barrier()
        off = 0
        while off < rpt:
            step = min(128, rpt - off)
            pltpu.sync_copy(acc.at[pl.ds(s * rpt + off, step)],
                            agg_h.at[c, pl.ds(s * rpt + off, step)])
            off += step
        plsc.subcore_barrier()

    return body(hm, e, src_r, dst_r, zeros_hbm)


def _update2_body(h_ref, agg_ref, wa_ref, wb_ref, b1_ref, w2_ref, b2_ref, o_ref):
    h = h_ref[...]
    agg = agg_ref[0] + agg_ref[1]
    u = jax.nn.silu(
        jnp.dot(h, wa_ref[...], preferred_element_type=jnp.float32)
        + jnp.dot(agg, wb_ref[...], preferred_element_type=jnp.float32)
        + b1_ref[...])
    o_ref[...] = h + jnp.dot(u, w2_ref[...], preferred_element_type=jnp.float32) + b2_ref[...]


def _update2(h, agg2, p, block=1024):
    n = h.shape[0]
    return pl.pallas_call(
        _update2_body,
        grid=(pl.cdiv(n, block),),
        in_specs=[
            pl.BlockSpec((block, HID), lambda i: (i, 0)),
            pl.BlockSpec((2, block, HID), lambda i: (0, i, 0)),
            pl.BlockSpec((HID, HID), lambda i: (0, 0)),
            pl.BlockSpec((HID, HID), lambda i: (0, 0)),
            pl.BlockSpec((1, HID), lambda i: (0, 0)),
            pl.BlockSpec((HID, HID), lambda i: (0, 0)),
            pl.BlockSpec((1, HID), lambda i: (0, 0)),
        ],
        out_specs=pl.BlockSpec((block, HID), lambda i: (i, 0)),
        out_shape=jax.ShapeDtypeStruct((n, HID), jnp.float32),
    )(h, agg2, p['Wu1'][:HID], p['Wu1'][HID:],
      p['bu1'].reshape(1, -1), p['Wu2'], p['bu2'].reshape(1, -1))


def _prep_edges(edge_index, n_dst, blk=EBLK):
    src = edge_index[0]
    dst = edge_index[1]
    e_num = src.shape[0]
    e_pad = _pad_to(e_num, blk)
    src_r = jnp.pad(src, (0, e_pad - e_num)).astype(jnp.int32).reshape(e_pad // 128, 128)
    dst_r = jnp.pad(dst, (0, e_pad - e_num),
                    constant_values=n_dst).astype(jnp.int32).reshape(e_pad // 128, 128)
    return src_r, dst_r, e_pad


def _round_sc(p, h_src, h_dst, src_r, dst_r, e, zeros_hbm, n_pad, nblk):
    hm = _affine_silu(h_src, p['Wm'], p['bm'])
    agg = _edge_sc(hm, e, src_r, dst_r, zeros_hbm, n_pad, nblk)
    return _update(h_dst, agg, p)


def _round_jnp(p, h_src, h_dst, src, dst, e, n_dst):
    hm = jax.nn.silu(h_src @ p['Wm'] + p['bm'])
    m = jnp.take(hm, src, axis=0) * e
    agg = jax.ops.segment_sum(m, dst, num_segments=n_dst)
    u = jnp.concatenate([h_dst, agg], axis=-1)
    return h_dst + jax.nn.silu(u @ p['Wu1'] + p['bu1']) @ p['Wu2'] + p['bu2']


def kernel(z, bond_edge_index, bond_edge_attr, aq_edge_index, aq_edge_attr,
           qq_edge_index, qq_edge_attr, n_query, params):
    p = params
    zeros_hbm = jnp.zeros((128, HID), jnp.float32)
    h_atom = jnp.take(p['emb'], z, axis=0)

    n_pad_b = _pad_to(N_ATOM + 1, 1024)
    src_b, dst_b, e_pad_b = _prep_edges(bond_edge_index, N_ATOM, blk=32 * C_EDGE)
    attr_b = jnp.pad(bond_edge_attr, ((0, e_pad_b - bond_edge_attr.shape[0]), (0, 0)))
    e_b = _affine_silu(attr_b, p['bond']['Wr'], p['bond']['br'])
    nblk32_b = e_pad_b // (32 * C_EDGE)
    for _ in range(2):
        hm_b = _affine_silu(h_atom, p['bond']['Wm'], p['bond']['bm'])
        agg2_b = _edge_sc_partial(hm_b, e_b, src_b, dst_b, zeros_hbm,
                                  n_pad_b, nblk32_b)
        h_atom = _update2(h_atom, agg2_b, p['bond'])

    n_pad_q = _pad_to(N_QUERY + 1, 1024)
    src_a, dst_a, e_pad_a = _prep_edges(aq_edge_index, N_QUERY)
    attr_a = jnp.pad(aq_edge_attr, ((0, e_pad_a - aq_edge_attr.shape[0]), (0, 0)))
    e_a = _affine_silu(attr_a, p['aq']['Wr'], p['aq']['br'])
    nblk_a = e_pad_a // EBLK
    h_query = jnp.zeros((N_QUERY, HID), jnp.float32)
    for _ in range(3):
        h_query = _round_sc(p['aq'], h_atom, h_query, src_a, dst_a, e_a,
                            zeros_hbm, n_pad_q, nblk_a)

    src_q, dst_q, e_pad_q = _prep_edges(qq_edge_index, N_QUERY)
    attr_q = jnp.pad(qq_edge_attr, ((0, e_pad_q - qq_edge_attr.shape[0]), (0, 0)))
    e_q = _affine_silu(attr_q, p['qq']['Wr'], p['qq']['br'])
    nblk_q = e_pad_q // EBLK
    for _ in range(2):
        h_query = _round_sc(p['qq'], h_query, h_query, src_q, dst_q, e_q,
                            zeros_hbm, n_pad_q, nblk_q)

    return _head(h_query, p).reshape(N_QUERY)


# speedup vs baseline: 1.4919x; 1.3366x over previous
import functools

import jax
import jax.numpy as jnp
from jax import lax
from jax.experimental import pallas as pl
from jax.experimental.pallas import tpu as pltpu
from jax.experimental.pallas import tpu_sc as plsc

HID = 128
N_ATOM = 10000
N_QUERY = 50000
C_EDGE = 128
EBLK = 16 * C_EDGE


def _pad_to(n, m):
    return ((n + m - 1) // m) * m


def _affine_silu_body(x_ref, w_ref, b_ref, o_ref):
    o_ref[...] = jax.nn.silu(
        jnp.dot(x_ref[...], w_ref[...], preferred_element_type=jnp.float32)
        + b_ref[...])


def _affine_silu(x, w, b, block=1024):
    n, k = x.shape
    m = w.shape[1]
    return pl.pallas_call(
        _affine_silu_body,
        grid=(pl.cdiv(n, block),),
        in_specs=[
            pl.BlockSpec((block, k), lambda i: (i, 0)),
            pl.BlockSpec((k, m), lambda i: (0, 0)),
            pl.BlockSpec((1, m), lambda i: (0, 0)),
        ],
        out_specs=pl.BlockSpec((block, m), lambda i: (i, 0)),
        out_shape=jax.ShapeDtypeStruct((n, m), jnp.float32),
    )(x, w, b.reshape(1, -1))


def _update_body(h_ref, agg_ref, wa_ref, wb_ref, b1_ref, w2_ref, b2_ref, o_ref):
    h = h_ref[...]
    u = jax.nn.silu(
        jnp.dot(h, wa_ref[...], preferred_element_type=jnp.float32)
        + jnp.dot(agg_ref[...], wb_ref[...], preferred_element_type=jnp.float32)
        + b1_ref[...])
    o_ref[...] = h + jnp.dot(u, w2_ref[...], preferred_element_type=jnp.float32) + b2_ref[...]


def _update(h, agg, p, block=1024):
    n = h.shape[0]
    return pl.pallas_call(
        _update_body,
        grid=(pl.cdiv(n, block),),
        in_specs=[
            pl.BlockSpec((block, HID), lambda i: (i, 0)),
            pl.BlockSpec((block, HID), lambda i: (i, 0)),
            pl.BlockSpec((HID, HID), lambda i: (0, 0)),
            pl.BlockSpec((HID, HID), lambda i: (0, 0)),
            pl.BlockSpec((1, HID), lambda i: (0, 0)),
            pl.BlockSpec((HID, HID), lambda i: (0, 0)),
            pl.BlockSpec((1, HID), lambda i: (0, 0)),
        ],
        out_specs=pl.BlockSpec((block, HID), lambda i: (i, 0)),
        out_shape=jax.ShapeDtypeStruct((n, HID), jnp.float32),
    )(h, agg, p['Wu1'][:HID], p['Wu1'][HID:],
      p['bu1'].reshape(1, -1), p['Wu2'], p['bu2'].reshape(1, -1))


def _head_body(h_ref, w1_ref, b1_ref, w2_ref, b2_ref, o_ref):
    t = jax.nn.silu(
        jnp.dot(h_ref[...], w1_ref[...], preferred_element_type=jnp.float32)
        + b1_ref[...])
    o_ref[...] = jnp.dot(t, w2_ref[...], preferred_element_type=jnp.float32) + b2_ref[...]


def _head(h, params, block=1024):
    n = h.shape[0]
    hh = HID // 2
    return pl.pallas_call(
        _head_body,
        grid=(pl.cdiv(n, block),),
        in_specs=[
            pl.BlockSpec((block, HID), lambda i: (i, 0)),
            pl.BlockSpec((HID, hh), lambda i: (0, 0)),
            pl.BlockSpec((1, hh), lambda i: (0, 0)),
            pl.BlockSpec((hh, 1), lambda i: (0, 0)),
            pl.BlockSpec((1, 1), lambda i: (0, 0)),
        ],
        out_specs=pl.BlockSpec((block, 1), lambda i: (i, 0)),
        out_shape=jax.ShapeDtypeStruct((n, 1), jnp.float32),
    )(h, params['Wh1'], params['bh1'].reshape(1, -1),
      params['Wh2'], params['bh2'].reshape(1, -1))


_TILE_SCRATCH = [
    pltpu.VMEM((2, 128), jnp.int32),
    pltpu.VMEM((2, 128), jnp.int32),
    pltpu.VMEM((2, 2, 64), jnp.int32),
    pltpu.VMEM((2, 64, HID), jnp.float32),
    pltpu.VMEM((64, HID), jnp.float32),
    pltpu.SemaphoreType.DMA,
    pltpu.SemaphoreType.DMA,
    pltpu.SemaphoreType.DMA,
    pltpu.SemaphoreType.DMA,
    pltpu.SemaphoreType.DMA,
]


def _load_idx(src_h, dst_h, src_v, dst_v, sidx_v, base, q, lo, R):
    pltpu.sync_copy(src_h.at[pl.ds(base, 1)], src_v.at[pl.ds(q, 1)])
    pltpu.sync_copy(dst_h.at[pl.ds(base, 1)], dst_v.at[pl.ds(q, 1)])
    for h in range(2):
        for k in range(4):
            d = dst_v[q, pl.ds(h * 64 + k * 16, 16)]
            local = d - lo
            ok = (local >= 0) & (local < R)
            sidx_v[q, h, pl.ds(k * 16, 16)] = jnp.where(ok, local, R)


def _edge_pass(hm_h, e_h, src_h, dst_h, acc,
               src_v, dst_v, sidx_v, rows_v, e_v,
               g, sc, se, nblk, base_fn, lo, R):
    base0 = base_fn(0)
    _load_idx(src_h, dst_h, src_v, dst_v, sidx_v, base0, 0, lo, R)
    for h in range(2):
        pltpu.async_copy(hm_h.at[src_v.at[0, pl.ds(h * 64, 64)]],
                         rows_v.at[h], g[h])
    pltpu.async_copy(e_h.at[pl.ds(base0 * 128, 64)], e_v, se)

    def blk(b, carry):
        base = base_fn(b)
        q = b % 2
        qn = (b + 1) % 2
        nxt = b + 1 < nblk

        @pl.when(nxt)
        def _():
            _load_idx(src_h, dst_h, src_v, dst_v, sidx_v,
                      base_fn(b + 1), qn, lo, R)

        for h in range(2):
            pltpu.make_async_copy(hm_h.at[pl.ds(0, 64)], rows_v.at[h], g[h]).wait()
            pltpu.make_async_copy(e_h.at[pl.ds(0, 64)], e_v, se).wait()

            def mul(r, carry2):
                for k in range(8):
                    rows_v[h, r, pl.ds(k * 16, 16)] = (
                        rows_v[h, r, pl.ds(k * 16, 16)] * e_v[r, pl.ds(k * 16, 16)])
                return carry2
            lax.fori_loop(0, 64, mul, 0)
            if h == 0:
                pltpu.async_copy(e_h.at[pl.ds(base * 128 + 64, 64)], e_v, se)
            else:
                @pl.when(nxt)
                def _():
                    pltpu.async_copy(e_h.at[pl.ds(base_fn(b + 1) * 128, 64)],
                                     e_v, se)
            pltpu.async_copy(rows_v.at[h], acc.at[sidx_v.at[q, h]],
                             sc[h], add=True)

        @pl.when(nxt)
        def _():
            for h in range(2):
                pltpu.make_async_copy(hm_h.at[pl.ds(0, 64)],
                                      rows_v.at[h], sc[h]).wait()
                pltpu.async_copy(hm_h.at[src_v.at[qn, pl.ds(h * 64, 64)]],
                                 rows_v.at[h], g[h])
        return carry
    lax.fori_loop(0, nblk, blk, 0)
    for h in range(2):
        pltpu.make_async_copy(hm_h.at[pl.ds(0, 64)], rows_v.at[h], sc[h]).wait()

def _edge_sc(hm, e, src_r, dst_r, zeros_hbm, n_pad, nblk):
    R = n_pad // 4
    acc_rows = R + 128
    rpt_z = acc_rows // 16
    rpt_f = R // 16
    mesh = plsc.VectorSubcoreMesh(core_axis_name="c", subcore_axis_name="s")

    @functools.partial(
        pl.kernel, mesh=mesh,
        out_type=jax.ShapeDtypeStruct((n_pad, HID), jnp.float32),
        scratch_types=[pltpu.VMEM_SHARED((acc_rows, HID), jnp.float32)]
        + _TILE_SCRATCH,
    )
    def body(hm_h, e_h, src_h, dst_h, z_h, agg_h,
             acc, src_v, dst_v, sidx_v, rows_v, e_v, g0, g1, s0, s1, se):
        c = lax.axis_index("c")
        s = lax.axis_index("s")
        for p in range(2):
            chunk = 2 * p + c
            lo = chunk * R
            off = 0
            while off < rpt_z:
                step = min(128, rpt_z - off)
                pltpu.sync_copy(z_h.at[pl.ds(0, step)],
                                acc.at[pl.ds(s * rpt_z + off, step)])
                off += step
            plsc.subcore_barrier()
            _edge_pass(hm_h, e_h, src_h, dst_h, acc,
                       src_v, dst_v, sidx_v, rows_v, e_v,
                       (g0, g1), (s0, s1), se,
                       nblk, lambda b: s * nblk + b, lo, R)
            plsc.subcore_barrier()
            off = 0
            while off < rpt_f:
                step = min(128, rpt_f - off)
                pltpu.sync_copy(acc.at[pl.ds(s * rpt_f + off, step)],
                                agg_h.at[pl.ds(lo + s * rpt_f + off, step)])
                off += step
            plsc.subcore_barrier()

    return body(hm, e, src_r, dst_r, zeros_hbm)


def _edge_sc_partial(hm, e, src_r, dst_r, zeros_hbm, n_pad, nblk32):
    rpt = n_pad // 16
    mesh = plsc.VectorSubcoreMesh(core_axis_name="c", subcore_axis_name="s")

    @functools.partial(
        pl.kernel, mesh=mesh,
        out_type=jax.ShapeDtypeStruct((2, n_pad, HID), jnp.float32),
        scratch_types=[pltpu.VMEM_SHARED((n_pad + 128, HID), jnp.float32)]
        + _TILE_SCRATCH,
    )
    def body(hm_h, e_h, src_h, dst_h, z_h, agg_h,
             acc, src_v, dst_v, sidx_v, rows_v, e_v, g0, g1, s0, s1, se):
        c = lax.axis_index("c")
        s = lax.axis_index("s")
        rpt_z = (n_pad + 128) // 16
        off = 0
        while off < rpt_z:
            step = min(128, rpt_z - off)
            pltpu.sync_copy(z_h.at[pl.ds(0, step)],
                            acc.at[pl.ds(s * rpt_z + off, step)])
            off += step
        plsc.subcore_barrier()
        _edge_pass(hm_h, e_h, src_h, dst_h, acc,
                   src_v, dst_v, sidx_v, rows_v, e_v,
                   (g0, g1), (s0, s1), se,
                   nblk32, lambda b: (c * 16 + s) * nblk32 + b, 0, n_pad)
        plsc.subcore_barrier()
        off = 0
        while off < rpt:
            step = min(128, rpt - off)
            pltpu.sync_copy(acc.at[pl.ds(s * rpt + off, step)],
                            agg_h.at[c, pl.ds(s * rpt + off, step)])
            off += step
        plsc.subcore_barrier()

    return body(hm, e, src_r, dst_r, zeros_hbm)


def _update2_body(h_ref, agg_ref, wa_ref, wb_ref, b1_ref, w2_ref, b2_ref, o_ref):
    h = h_ref[...]
    agg = agg_ref[0] + agg_ref[1]
    u = jax.nn.silu(
        jnp.dot(h, wa_ref[...], preferred_element_type=jnp.float32)
        + jnp.dot(agg, wb_ref[...], preferred_element_type=jnp.float32)
        + b1_ref[...])
    o_ref[...] = h + jnp.dot(u, w2_ref[...], preferred_element_type=jnp.float32) + b2_ref[...]


def _update2(h, agg2, p, block=1024):
    n = h.shape[0]
    return pl.pallas_call(
        _update2_body,
        grid=(pl.cdiv(n, block),),
        in_specs=[
            pl.BlockSpec((block, HID), lambda i: (i, 0)),
            pl.BlockSpec((2, block, HID), lambda i: (0, i, 0)),
            pl.BlockSpec((HID, HID), lambda i: (0, 0)),
            pl.BlockSpec((HID, HID), lambda i: (0, 0)),
            pl.BlockSpec((1, HID), lambda i: (0, 0)),
            pl.BlockSpec((HID, HID), lambda i: (0, 0)),
            pl.BlockSpec((1, HID), lambda i: (0, 0)),
        ],
        out_specs=pl.BlockSpec((block, HID), lambda i: (i, 0)),
        out_shape=jax.ShapeDtypeStruct((n, HID), jnp.float32),
    )(h, agg2, p['Wu1'][:HID], p['Wu1'][HID:],
      p['bu1'].reshape(1, -1), p['Wu2'], p['bu2'].reshape(1, -1))


def _prep_edges(edge_index, n_dst, blk=EBLK):
    src = edge_index[0]
    dst = edge_index[1]
    e_num = src.shape[0]
    e_pad = _pad_to(e_num, blk)
    src_r = jnp.pad(src, (0, e_pad - e_num)).astype(jnp.int32).reshape(e_pad // 128, 128)
    dst_r = jnp.pad(dst, (0, e_pad - e_num),
                    constant_values=n_dst).astype(jnp.int32).reshape(e_pad // 128, 128)
    return src_r, dst_r, e_pad


def _round_sc(p, h_src, h_dst, src_r, dst_r, e, zeros_hbm, n_pad, nblk):
    hm = _affine_silu(h_src, p['Wm'], p['bm'])
    agg = _edge_sc(hm, e, src_r, dst_r, zeros_hbm, n_pad, nblk)
    return _update(h_dst, agg, p)


def _round_jnp(p, h_src, h_dst, src, dst, e, n_dst):
    hm = jax.nn.silu(h_src @ p['Wm'] + p['bm'])
    m = jnp.take(hm, src, axis=0) * e
    agg = jax.ops.segment_sum(m, dst, num_segments=n_dst)
    u = jnp.concatenate([h_dst, agg], axis=-1)
    return h_dst + jax.nn.silu(u @ p['Wu1'] + p['bu1']) @ p['Wu2'] + p['bu2']


def kernel(z, bond_edge_index, bond_edge_attr, aq_edge_index, aq_edge_attr,
           qq_edge_index, qq_edge_attr, n_query, params):
    p = params
    zeros_hbm = jnp.zeros((128, HID), jnp.float32)
    h_atom = jnp.take(p['emb'], z, axis=0)

    n_pad_b = _pad_to(N_ATOM + 1, 1024)
    src_b, dst_b, e_pad_b = _prep_edges(bond_edge_index, N_ATOM, blk=32 * C_EDGE)
    attr_b = jnp.pad(bond_edge_attr, ((0, e_pad_b - bond_edge_attr.shape[0]), (0, 0)))
    e_b = _affine_silu(attr_b, p['bond']['Wr'], p['bond']['br'])
    nblk32_b = e_pad_b // (32 * C_EDGE)
    for _ in range(2):
        hm_b = _affine_silu(h_atom, p['bond']['Wm'], p['bond']['bm'])
        agg2_b = _edge_sc_partial(hm_b, e_b, src_b, dst_b, zeros_hbm,
                                  n_pad_b, nblk32_b)
        h_atom = _update2(h_atom, agg2_b, p['bond'])

    n_pad_q = _pad_to(N_QUERY + 1, 1024)
    src_a, dst_a, e_pad_a = _prep_edges(aq_edge_index, N_QUERY)
    attr_a = jnp.pad(aq_edge_attr, ((0, e_pad_a - aq_edge_attr.shape[0]), (0, 0)))
    e_a = _affine_silu(attr_a, p['aq']['Wr'], p['aq']['br'])
    nblk_a = e_pad_a // EBLK
    h_query = jnp.zeros((N_QUERY, HID), jnp.float32)
    for _ in range(3):
        h_query = _round_sc(p['aq'], h_atom, h_query, src_a, dst_a, e_a,
                            zeros_hbm, n_pad_q, nblk_a)

    src_q, dst_q, e_pad_q = _prep_edges(qq_edge_index, N_QUERY)
    attr_q = jnp.pad(qq_edge_attr, ((0, e_pad_q - qq_edge_attr.shape[0]), (0, 0)))
    e_q = _affine_silu(attr_q, p['qq']['Wr'], p['qq']['br'])
    nblk_q = e_pad_q // EBLK
    for _ in range(2):
        h_query = _round_sc(p['qq'], h_query, h_query, src_q, dst_q, e_q,
                            zeros_hbm, n_pad_q, nblk_q)

    return _head(h_query, p).reshape(N_QUERY)


# speedup vs baseline: 1.5604x; 1.0459x over previous
import functools

import jax
import jax.numpy as jnp
from jax import lax
from jax.experimental import pallas as pl
from jax.experimental.pallas import tpu as pltpu
from jax.experimental.pallas import tpu_sc as plsc

HID = 128
N_ATOM = 10000
N_QUERY = 50000
C_EDGE = 128
EBLK = 16 * C_EDGE


def _pad_to(n, m):
    return ((n + m - 1) // m) * m


def _affine_silu_body(x_ref, w_ref, b_ref, o_ref):
    o_ref[...] = jax.nn.silu(
        jnp.dot(x_ref[...], w_ref[...], preferred_element_type=jnp.float32)
        + b_ref[...])


def _affine_silu(x, w, b, block=1024):
    n, k = x.shape
    m = w.shape[1]
    return pl.pallas_call(
        _affine_silu_body,
        grid=(pl.cdiv(n, block),),
        in_specs=[
            pl.BlockSpec((block, k), lambda i: (i, 0)),
            pl.BlockSpec((k, m), lambda i: (0, 0)),
            pl.BlockSpec((1, m), lambda i: (0, 0)),
        ],
        out_specs=pl.BlockSpec((block, m), lambda i: (i, 0)),
        out_shape=jax.ShapeDtypeStruct((n, m), jnp.float32),
    )(x, w, b.reshape(1, -1))


def _update_body(h_ref, agg_ref, wa_ref, wb_ref, b1_ref, w2_ref, b2_ref, o_ref):
    h = h_ref[...]
    u = jax.nn.silu(
        jnp.dot(h, wa_ref[...], preferred_element_type=jnp.float32)
        + jnp.dot(agg_ref[...], wb_ref[...], preferred_element_type=jnp.float32)
        + b1_ref[...])
    o_ref[...] = h + jnp.dot(u, w2_ref[...], preferred_element_type=jnp.float32) + b2_ref[...]


def _update(h, agg, p, block=1024):
    n = h.shape[0]
    return pl.pallas_call(
        _update_body,
        grid=(pl.cdiv(n, block),),
        in_specs=[
            pl.BlockSpec((block, HID), lambda i: (i, 0)),
            pl.BlockSpec((block, HID), lambda i: (i, 0)),
            pl.BlockSpec((HID, HID), lambda i: (0, 0)),
            pl.BlockSpec((HID, HID), lambda i: (0, 0)),
            pl.BlockSpec((1, HID), lambda i: (0, 0)),
            pl.BlockSpec((HID, HID), lambda i: (0, 0)),
            pl.BlockSpec((1, HID), lambda i: (0, 0)),
        ],
        out_specs=pl.BlockSpec((block, HID), lambda i: (i, 0)),
        out_shape=jax.ShapeDtypeStruct((n, HID), jnp.float32),
    )(h, agg, p['Wu1'][:HID], p['Wu1'][HID:],
      p['bu1'].reshape(1, -1), p['Wu2'], p['bu2'].reshape(1, -1))


def _head_body(h_ref, w1_ref, b1_ref, w2_ref, b2_ref, o_ref):
    t = jax.nn.silu(
        jnp.dot(h_ref[...], w1_ref[...], preferred_element_type=jnp.float32)
        + b1_ref[...])
    o_ref[...] = jnp.dot(t, w2_ref[...], preferred_element_type=jnp.float32) + b2_ref[...]


def _head(h, params, block=1024):
    n = h.shape[0]
    hh = HID // 2
    return pl.pallas_call(
        _head_body,
        grid=(pl.cdiv(n, block),),
        in_specs=[
            pl.BlockSpec((block, HID), lambda i: (i, 0)),
            pl.BlockSpec((HID, hh), lambda i: (0, 0)),
            pl.BlockSpec((1, hh), lambda i: (0, 0)),
            pl.BlockSpec((hh, 1), lambda i: (0, 0)),
            pl.BlockSpec((1, 1), lambda i: (0, 0)),
        ],
        out_specs=pl.BlockSpec((block, 1), lambda i: (i, 0)),
        out_shape=jax.ShapeDtypeStruct((n, 1), jnp.float32),
    )(h, params['Wh1'], params['bh1'].reshape(1, -1),
      params['Wh2'], params['bh2'].reshape(1, -1))


_TILE_SCRATCH = [
    pltpu.VMEM((2, 128), jnp.int32),
    pltpu.VMEM((2, 128), jnp.int32),
    pltpu.VMEM((2, 2, 64), jnp.int32),
    pltpu.VMEM((2, 64, HID), jnp.float32),
    pltpu.VMEM((64, HID), jnp.float32),
    pltpu.SemaphoreType.DMA,
    pltpu.SemaphoreType.DMA,
    pltpu.SemaphoreType.DMA,
    pltpu.SemaphoreType.DMA,
    pltpu.SemaphoreType.DMA,
    pltpu.SemaphoreType.DMA,
]


def _issue_idx(src_h, dst_h, src_v, dst_v, base, q, si):
    pltpu.async_copy(src_h.at[pl.ds(base, 1)], src_v.at[pl.ds(q, 1)], si)
    pltpu.async_copy(dst_h.at[pl.ds(base, 1)], dst_v.at[pl.ds(q, 1)], si)


def _wait_idx(src_h, src_v, dst_v, q, si):
    pltpu.make_async_copy(src_h.at[pl.ds(0, 1)], src_v.at[pl.ds(q, 1)], si).wait()
    pltpu.make_async_copy(src_h.at[pl.ds(0, 1)], dst_v.at[pl.ds(q, 1)], si).wait()


def _calc_sidx(dst_v, sidx_v, q, lo, R):
    for h in range(2):
        for k in range(4):
            d = dst_v[q, pl.ds(h * 64 + k * 16, 16)]
            local = d - lo
            ok = (local >= 0) & (local < R)
            sidx_v[q, h, pl.ds(k * 16, 16)] = jnp.where(ok, local, R)


def _edge_pass(hm_h, e_h, src_h, dst_h, acc,
               src_v, dst_v, sidx_v, rows_v, e_v,
               g, sc, se, si, nblk, base_fn, lo, R):
    base0 = base_fn(0)
    pltpu.sync_copy(src_h.at[pl.ds(base0, 1)], src_v.at[pl.ds(0, 1)])
    pltpu.sync_copy(dst_h.at[pl.ds(base0, 1)], dst_v.at[pl.ds(0, 1)])
    _calc_sidx(dst_v, sidx_v, 0, lo, R)
    if nblk > 1:
        _issue_idx(src_h, dst_h, src_v, dst_v, base_fn(1), 1, si)
    for h in range(2):
        pltpu.async_copy(hm_h.at[src_v.at[0, pl.ds(h * 64, 64)]],
                         rows_v.at[h], g[h])
    pltpu.async_copy(e_h.at[pl.ds(base0 * 128, 64)], e_v, se)

    def blk(b, carry):
        base = base_fn(b)
        q = b % 2
        qn = (b + 1) % 2
        nxt = b + 1 < nblk

        @pl.when(nxt)
        def _():
            _wait_idx(src_h, src_v, dst_v, qn, si)
            _calc_sidx(dst_v, sidx_v, qn, lo, R)

        for h in range(2):
            pltpu.make_async_copy(hm_h.at[pl.ds(0, 64)], rows_v.at[h], g[h]).wait()
            pltpu.make_async_copy(e_h.at[pl.ds(0, 64)], e_v, se).wait()

            def mul(r, carry2):
                r4 = r * 4
                for rr in range(4):
                    for k in range(8):
                        rows_v[h, r4 + rr, pl.ds(k * 16, 16)] = (
                            rows_v[h, r4 + rr, pl.ds(k * 16, 16)]
                            * e_v[r4 + rr, pl.ds(k * 16, 16)])
                return carry2
            lax.fori_loop(0, 16, mul, 0)
            if h == 0:
                pltpu.async_copy(e_h.at[pl.ds(base * 128 + 64, 64)], e_v, se)
            else:
                @pl.when(nxt)
                def _():
                    pltpu.async_copy(e_h.at[pl.ds(base_fn(b + 1) * 128, 64)],
                                     e_v, se)
            pltpu.async_copy(rows_v.at[h], acc.at[sidx_v.at[q, h]],
                             sc[h], add=True)

        @pl.when(b + 2 < nblk)
        def _():
            _issue_idx(src_h, dst_h, src_v, dst_v, base_fn(b + 2), q, si)

        @pl.when(nxt)
        def _():
            for h in range(2):
                pltpu.make_async_copy(hm_h.at[pl.ds(0, 64)],
                                      rows_v.at[h], sc[h]).wait()
                pltpu.async_copy(hm_h.at[src_v.at[qn, pl.ds(h * 64, 64)]],
                                 rows_v.at[h], g[h])
        return carry
    lax.fori_loop(0, nblk, blk, 0)
    for h in range(2):
        pltpu.make_async_copy(hm_h.at[pl.ds(0, 64)], rows_v.at[h], sc[h]).wait()

def _edge_sc(hm, e, src_r, dst_r, zeros_hbm, n_pad, nblk):
    R = n_pad // 4
    acc_rows = R + 128
    rpt_z = acc_rows // 16
    rpt_f = R // 16
    mesh = plsc.VectorSubcoreMesh(core_axis_name="c", subcore_axis_name="s")

    @functools.partial(
        pl.kernel, mesh=mesh,
        out_type=jax.ShapeDtypeStruct((n_pad, HID), jnp.float32),
        scratch_types=[pltpu.VMEM_SHARED((acc_rows, HID), jnp.float32)]
        + _TILE_SCRATCH,
    )
    def body(hm_h, e_h, src_h, dst_h, z_h, agg_h,
             acc, src_v, dst_v, sidx_v, rows_v, e_v, g0, g1, s0, s1, se, si):
        c = lax.axis_index("c")
        s = lax.axis_index("s")
        for p in range(2):
            chunk = 2 * p + c
            lo = chunk * R
            off = 0
            while off < rpt_z:
                step = min(128, rpt_z - off)
                pltpu.sync_copy(z_h.at[pl.ds(0, step)],
                                acc.at[pl.ds(s * rpt_z + off, step)])
                off += step
            plsc.subcore_barrier()
            _edge_pass(hm_h, e_h, src_h, dst_h, acc,
                       src_v, dst_v, sidx_v, rows_v, e_v,
                       (g0, g1), (s0, s1), se, si,
                       nblk, lambda b: s * nblk + b, lo, R)
            plsc.subcore_barrier()
            off = 0
            while off < rpt_f:
                step = min(128, rpt_f - off)
                pltpu.sync_copy(acc.at[pl.ds(s * rpt_f + off, step)],
                                agg_h.at[pl.ds(lo + s * rpt_f + off, step)])
                off += step
            plsc.subcore_barrier()

    return body(hm, e, src_r, dst_r, zeros_hbm)


def _edge_sc_partial(hm, e, src_r, dst_r, zeros_hbm, n_pad, nblk32):
    rpt = n_pad // 16
    mesh = plsc.VectorSubcoreMesh(core_axis_name="c", subcore_axis_name="s")

    @functools.partial(
        pl.kernel, mesh=mesh,
        out_type=jax.ShapeDtypeStruct((2, n_pad, HID), jnp.float32),
        scratch_types=[pltpu.VMEM_SHARED((n_pad + 128, HID), jnp.float32)]
        + _TILE_SCRATCH,
    )
    def body(hm_h, e_h, src_h, dst_h, z_h, agg_h,
             acc, src_v, dst_v, sidx_v, rows_v, e_v, g0, g1, s0, s1, se, si):
        c = lax.axis_index("c")
        s = lax.axis_index("s")
        rpt_z = (n_pad + 128) // 16
        off = 0
        while off < rpt_z:
            step = min(128, rpt_z - off)
            pltpu.sync_copy(z_h.at[pl.ds(0, step)],
                            acc.at[pl.ds(s * rpt_z + off, step)])
            off += step
        plsc.subcore_barrier()
        _edge_pass(hm_h, e_h, src_h, dst_h, acc,
                   src_v, dst_v, sidx_v, rows_v, e_v,
                   (g0, g1), (s0, s1), se, si,
                   nblk32, lambda b: (c * 16 + s) * nblk32 + b, 0, n_pad)
        plsc.subcore_barrier()
        off = 0
        while off < rpt:
            step = min(128, rpt - off)
            pltpu.sync_copy(acc.at[pl.ds(s * rpt + off, step)],
                            agg_h.at[c, pl.ds(s * rpt + off, step)])
            off += step
        plsc.subcore_barrier()

    return body(hm, e, src_r, dst_r, zeros_hbm)


def _update2_body(h_ref, agg_ref, wa_ref, wb_ref, b1_ref, w2_ref, b2_ref, o_ref):
    h = h_ref[...]
    agg = agg_ref[0] + agg_ref[1]
    u = jax.nn.silu(
        jnp.dot(h, wa_ref[...], preferred_element_type=jnp.float32)
        + jnp.dot(agg, wb_ref[...], preferred_element_type=jnp.float32)
        + b1_ref[...])
    o_ref[...] = h + jnp.dot(u, w2_ref[...], preferred_element_type=jnp.float32) + b2_ref[...]


def _update2(h, agg2, p, block=1024):
    n = h.shape[0]
    return pl.pallas_call(
        _update2_body,
        grid=(pl.cdiv(n, block),),
        in_specs=[
            pl.BlockSpec((block, HID), lambda i: (i, 0)),
            pl.BlockSpec((2, block, HID), lambda i: (0, i, 0)),
            pl.BlockSpec((HID, HID), lambda i: (0, 0)),
            pl.BlockSpec((HID, HID), lambda i: (0, 0)),
            pl.BlockSpec((1, HID), lambda i: (0, 0)),
            pl.BlockSpec((HID, HID), lambda i: (0, 0)),
            pl.BlockSpec((1, HID), lambda i: (0, 0)),
        ],
        out_specs=pl.BlockSpec((block, HID), lambda i: (i, 0)),
        out_shape=jax.ShapeDtypeStruct((n, HID), jnp.float32),
    )(h, agg2, p['Wu1'][:HID], p['Wu1'][HID:],
      p['bu1'].reshape(1, -1), p['Wu2'], p['bu2'].reshape(1, -1))


def _prep_edges(edge_index, n_dst, blk=EBLK):
    src = edge_index[0]
    dst = edge_index[1]
    e_num = src.shape[0]
    e_pad = _pad_to(e_num, blk)
    src_r = jnp.pad(src, (0, e_pad - e_num)).astype(jnp.int32).reshape(e_pad // 128, 128)
    dst_r = jnp.pad(dst, (0, e_pad - e_num),
                    constant_values=n_dst).astype(jnp.int32).reshape(e_pad // 128, 128)
    return src_r, dst_r, e_pad


def _round_sc(p, h_src, h_dst, src_r, dst_r, e, zeros_hbm, n_pad, nblk):
    hm = _affine_silu(h_src, p['Wm'], p['bm'])
    agg = _edge_sc(hm, e, src_r, dst_r, zeros_hbm, n_pad, nblk)
    return _update(h_dst, agg, p)


def _round_jnp(p, h_src, h_dst, src, dst, e, n_dst):
    hm = jax.nn.silu(h_src @ p['Wm'] + p['bm'])
    m = jnp.take(hm, src, axis=0) * e
    agg = jax.ops.segment_sum(m, dst, num_segments=n_dst)
    u = jnp.concatenate([h_dst, agg], axis=-1)
    return h_dst + jax.nn.silu(u @ p['Wu1'] + p['bu1']) @ p['Wu2'] + p['bu2']


def kernel(z, bond_edge_index, bond_edge_attr, aq_edge_index, aq_edge_attr,
           qq_edge_index, qq_edge_attr, n_query, params):
    p = params
    zeros_hbm = jnp.zeros((128, HID), jnp.float32)
    h_atom = jnp.take(p['emb'], z, axis=0)

    n_pad_b = _pad_to(N_ATOM + 1, 1024)
    src_b, dst_b, e_pad_b = _prep_edges(bond_edge_index, N_ATOM, blk=32 * C_EDGE)
    attr_b = jnp.pad(bond_edge_attr, ((0, e_pad_b - bond_edge_attr.shape[0]), (0, 0)))
    e_b = _affine_silu(attr_b, p['bond']['Wr'], p['bond']['br'])
    nblk32_b = e_pad_b // (32 * C_EDGE)
    for _ in range(2):
        hm_b = _affine_silu(h_atom, p['bond']['Wm'], p['bond']['bm'])
        agg2_b = _edge_sc_partial(hm_b, e_b, src_b, dst_b, zeros_hbm,
                                  n_pad_b, nblk32_b)
        h_atom = _update2(h_atom, agg2_b, p['bond'])

    n_pad_q = _pad_to(N_QUERY + 1, 1024)
    src_a, dst_a, e_pad_a = _prep_edges(aq_edge_index, N_QUERY)
    attr_a = jnp.pad(aq_edge_attr, ((0, e_pad_a - aq_edge_attr.shape[0]), (0, 0)))
    e_a = _affine_silu(attr_a, p['aq']['Wr'], p['aq']['br'])
    nblk_a = e_pad_a // EBLK
    h_query = jnp.zeros((N_QUERY, HID), jnp.float32)
    for _ in range(3):
        h_query = _round_sc(p['aq'], h_atom, h_query, src_a, dst_a, e_a,
                            zeros_hbm, n_pad_q, nblk_a)

    src_q, dst_q, e_pad_q = _prep_edges(qq_edge_index, N_QUERY)
    attr_q = jnp.pad(qq_edge_attr, ((0, e_pad_q - qq_edge_attr.shape[0]), (0, 0)))
    e_q = _affine_silu(attr_q, p['qq']['Wr'], p['qq']['br'])
    nblk_q = e_pad_q // EBLK
    for _ in range(2):
        h_query = _round_sc(p['qq'], h_query, h_query, src_q, dst_q, e_q,
                            zeros_hbm, n_pad_q, nblk_q)

    return _head(h_query, p).reshape(N_QUERY)


# speedup vs baseline: 1.5633x; 1.0019x over previous
import functools

import jax
import jax.numpy as jnp
from jax import lax
from jax.experimental import pallas as pl
from jax.experimental.pallas import tpu as pltpu
from jax.experimental.pallas import tpu_sc as plsc

HID = 128
N_ATOM = 10000
N_QUERY = 50000
C_EDGE = 128
EBLK = 16 * C_EDGE


def _pad_to(n, m):
    return ((n + m - 1) // m) * m


def _affine_silu_body(x_ref, w_ref, b_ref, o_ref):
    o_ref[...] = jax.nn.silu(
        jnp.dot(x_ref[...], w_ref[...], preferred_element_type=jnp.float32)
        + b_ref[...])


def _affine_silu(x, w, b, block=1024):
    n, k = x.shape
    m = w.shape[1]
    return pl.pallas_call(
        _affine_silu_body,
        grid=(pl.cdiv(n, block),),
        in_specs=[
            pl.BlockSpec((block, k), lambda i: (i, 0)),
            pl.BlockSpec((k, m), lambda i: (0, 0)),
            pl.BlockSpec((1, m), lambda i: (0, 0)),
        ],
        out_specs=pl.BlockSpec((block, m), lambda i: (i, 0)),
        out_shape=jax.ShapeDtypeStruct((n, m), jnp.float32),
    )(x, w, b.reshape(1, -1))


def _update_body(h_ref, agg_ref, wa_ref, wb_ref, b1_ref, w2_ref, b2_ref, o_ref):
    h = h_ref[...]
    u = jax.nn.silu(
        jnp.dot(h, wa_ref[...], preferred_element_type=jnp.float32)
        + jnp.dot(agg_ref[...], wb_ref[...], preferred_element_type=jnp.float32)
        + b1_ref[...])
    o_ref[...] = h + jnp.dot(u, w2_ref[...], preferred_element_type=jnp.float32) + b2_ref[...]


def _update(h, agg, p, block=1024):
    n = h.shape[0]
    return pl.pallas_call(
        _update_body,
        grid=(pl.cdiv(n, block),),
        in_specs=[
            pl.BlockSpec((block, HID), lambda i: (i, 0)),
            pl.BlockSpec((block, HID), lambda i: (i, 0)),
            pl.BlockSpec((HID, HID), lambda i: (0, 0)),
            pl.BlockSpec((HID, HID), lambda i: (0, 0)),
            pl.BlockSpec((1, HID), lambda i: (0, 0)),
            pl.BlockSpec((HID, HID), lambda i: (0, 0)),
            pl.BlockSpec((1, HID), lambda i: (0, 0)),
        ],
        out_specs=pl.BlockSpec((block, HID), lambda i: (i, 0)),
        out_shape=jax.ShapeDtypeStruct((n, HID), jnp.float32),
    )(h, agg, p['Wu1'][:HID], p['Wu1'][HID:],
      p['bu1'].reshape(1, -1), p['Wu2'], p['bu2'].reshape(1, -1))


def _head_body(h_ref, w1_ref, b1_ref, w2_ref, b2_ref, o_ref):
    t = jax.nn.silu(
        jnp.dot(h_ref[...], w1_ref[...], preferred_element_type=jnp.float32)
        + b1_ref[...])
    o_ref[...] = jnp.dot(t, w2_ref[...], preferred_element_type=jnp.float32) + b2_ref[...]


def _head(h, params, block=1024):
    n = h.shape[0]
    hh = HID // 2
    return pl.pallas_call(
        _head_body,
        grid=(pl.cdiv(n, block),),
        in_specs=[
            pl.BlockSpec((block, HID), lambda i: (i, 0)),
            pl.BlockSpec((HID, hh), lambda i: (0, 0)),
            pl.BlockSpec((1, hh), lambda i: (0, 0)),
            pl.BlockSpec((hh, 1), lambda i: (0, 0)),
            pl.BlockSpec((1, 1), lambda i: (0, 0)),
        ],
        out_specs=pl.BlockSpec((block, 1), lambda i: (i, 0)),
        out_shape=jax.ShapeDtypeStruct((n, 1), jnp.float32),
    )(h, params['Wh1'], params['bh1'].reshape(1, -1),
      params['Wh2'], params['bh2'].reshape(1, -1))


_TILE_SCRATCH = [
    pltpu.VMEM((2, 128), jnp.int32),
    pltpu.VMEM((2, 128), jnp.int32),
    pltpu.VMEM((2, 2, 64), jnp.int32),
    pltpu.VMEM((2, 64, HID), jnp.float32),
    pltpu.VMEM((64, HID), jnp.float32),
    pltpu.SemaphoreType.DMA,
    pltpu.SemaphoreType.DMA,
    pltpu.SemaphoreType.DMA,
    pltpu.SemaphoreType.DMA,
    pltpu.SemaphoreType.DMA,
    pltpu.SemaphoreType.DMA,
]


def _issue_idx(src_h, dst_h, src_v, dst_v, base, q, si):
    pltpu.async_copy(src_h.at[pl.ds(base, 1)], src_v.at[pl.ds(q, 1)], si)
    pltpu.async_copy(dst_h.at[pl.ds(base, 1)], dst_v.at[pl.ds(q, 1)], si)


def _wait_idx(src_h, src_v, dst_v, q, si):
    pltpu.make_async_copy(src_h.at[pl.ds(0, 1)], src_v.at[pl.ds(q, 1)], si).wait()
    pltpu.make_async_copy(src_h.at[pl.ds(0, 1)], dst_v.at[pl.ds(q, 1)], si).wait()


def _calc_sidx(dst_v, sidx_v, q, lo, R):
    for h in range(2):
        for k in range(4):
            d = dst_v[q, pl.ds(h * 64 + k * 16, 16)]
            local = d - lo
            ok = (local >= 0) & (local < R)
            sidx_v[q, h, pl.ds(k * 16, 16)] = jnp.where(ok, local, R)


def _edge_pass(hm_h, e_h, src_h, dst_h, acc,
               src_v, dst_v, sidx_v, rows_v, e_v,
               g, sc, se, si, nblk, base_fn, lo, R):
    base0 = base_fn(0)
    pltpu.sync_copy(src_h.at[pl.ds(base0, 1)], src_v.at[pl.ds(0, 1)])
    pltpu.sync_copy(dst_h.at[pl.ds(base0, 1)], dst_v.at[pl.ds(0, 1)])
    _calc_sidx(dst_v, sidx_v, 0, lo, R)
    if nblk > 1:
        _issue_idx(src_h, dst_h, src_v, dst_v, base_fn(1), 1, si)
    for h in range(2):
        pltpu.async_copy(hm_h.at[src_v.at[0, pl.ds(h * 64, 64)]],
                         rows_v.at[h], g[h])
    pltpu.async_copy(e_h.at[pl.ds(base0 * 128, 64)], e_v, se)

    def blk(b, carry):
        base = base_fn(b)
        q = b % 2
        qn = (b + 1) % 2
        nxt = b + 1 < nblk

        @pl.when(nxt)
        def _():
            _wait_idx(src_h, src_v, dst_v, qn, si)
            _calc_sidx(dst_v, sidx_v, qn, lo, R)

        for h in range(2):
            pltpu.make_async_copy(hm_h.at[pl.ds(0, 64)], rows_v.at[h], g[h]).wait()
            pltpu.make_async_copy(e_h.at[pl.ds(0, 64)], e_v, se).wait()

            @plsc.parallel_loop(0, 64, unroll=4)
            def mul(r):
                for k in range(8):
                    rows_v[h, r, pl.ds(k * 16, 16)] = (
                        rows_v[h, r, pl.ds(k * 16, 16)]
                        * e_v[r, pl.ds(k * 16, 16)])
            if h == 0:
                pltpu.async_copy(e_h.at[pl.ds(base * 128 + 64, 64)], e_v, se)
            else:
                @pl.when(nxt)
                def _():
                    pltpu.async_copy(e_h.at[pl.ds(base_fn(b + 1) * 128, 64)],
                                     e_v, se)
            pltpu.async_copy(rows_v.at[h], acc.at[sidx_v.at[q, h]],
                             sc[h], add=True)

        @pl.when(b + 2 < nblk)
        def _():
            _issue_idx(src_h, dst_h, src_v, dst_v, base_fn(b + 2), q, si)

        @pl.when(nxt)
        def _():
            for h in range(2):
                pltpu.make_async_copy(hm_h.at[pl.ds(0, 64)],
                                      rows_v.at[h], sc[h]).wait()
                pltpu.async_copy(hm_h.at[src_v.at[qn, pl.ds(h * 64, 64)]],
                                 rows_v.at[h], g[h])
        return carry
    lax.fori_loop(0, nblk, blk, 0)
    for h in range(2):
        pltpu.make_async_copy(hm_h.at[pl.ds(0, 64)], rows_v.at[h], sc[h]).wait()

def _edge_sc(hm, e, src_r, dst_r, zeros_hbm, n_pad, nblk):
    R = n_pad // 4
    acc_rows = R + 128
    rpt_z = acc_rows // 16
    rpt_f = R // 16
    mesh = plsc.VectorSubcoreMesh(core_axis_name="c", subcore_axis_name="s")

    @functools.partial(
        pl.kernel, mesh=mesh,
        out_type=jax.ShapeDtypeStruct((n_pad, HID), jnp.float32),
        scratch_types=[pltpu.VMEM_SHARED((acc_rows, HID), jnp.float32)]
        + _TILE_SCRATCH,
    )
    def body(hm_h, e_h, src_h, dst_h, z_h, agg_h,
             acc, src_v, dst_v, sidx_v, rows_v, e_v, g0, g1, s0, s1, se, si):
        c = lax.axis_index("c")
        s = lax.axis_index("s")
        for p in range(2):
            chunk = 2 * p + c
            lo = chunk * R
            off = 0
            while off < rpt_z:
                step = min(128, rpt_z - off)
                pltpu.sync_copy(z_h.at[pl.ds(0, step)],
                                acc.at[pl.ds(s * rpt_z + off, step)])
                off += step
            plsc.subcore_barrier()
            _edge_pass(hm_h, e_h, src_h, dst_h, acc,
                       src_v, dst_v, sidx_v, rows_v, e_v,
                       (g0, g1), (s0, s1), se, si,
                       nblk, lambda b: s * nblk + b, lo, R)
            plsc.subcore_barrier()
            off = 0
            while off < rpt_f:
                step = min(128, rpt_f - off)
                pltpu.sync_copy(acc.at[pl.ds(s * rpt_f + off, step)],
                                agg_h.at[pl.ds(lo + s * rpt_f + off, step)])
                off += step
            plsc.subcore_barrier()

    return body(hm, e, src_r, dst_r, zeros_hbm)


def _edge_sc_partial(hm, e, src_r, dst_r, zeros_hbm, n_pad, nblk32):
    rpt = n_pad // 16
    mesh = plsc.VectorSubcoreMesh(core_axis_name="c", subcore_axis_name="s")

    @functools.partial(
        pl.kernel, mesh=mesh,
        out_type=jax.ShapeDtypeStruct((2, n_pad, HID), jnp.float32),
        scratch_types=[pltpu.VMEM_SHARED((n_pad + 128, HID), jnp.float32)]
        + _TILE_SCRATCH,
    )
    def body(hm_h, e_h, src_h, dst_h, z_h, agg_h,
             acc, src_v, dst_v, sidx_v, rows_v, e_v, g0, g1, s0, s1, se, si):
        c = lax.axis_index("c")
        s = lax.axis_index("s")
        rpt_z = (n_pad + 128) // 16
        off = 0
        while off < rpt_z:
            step = min(128, rpt_z - off)
            pltpu.sync_copy(z_h.at[pl.ds(0, step)],
                            acc.at[pl.ds(s * rpt_z + off, step)])
            off += step
        plsc.subcore_barrier()
        _edge_pass(hm_h, e_h, src_h, dst_h, acc,
                   src_v, dst_v, sidx_v, rows_v, e_v,
                   (g0, g1), (s0, s1), se, si,
                   nblk32, lambda b: (c * 16 + s) * nblk32 + b, 0, n_pad)
        plsc.subcore_barrier()
        off = 0
        while off < rpt:
            step = min(128, rpt - off)
            pltpu.sync_copy(acc.at[pl.ds(s * rpt + off, step)],
                            agg_h.at[c, pl.ds(s * rpt + off, step)])
            off += step
        plsc.subcore_barrier()

    return body(hm, e, src_r, dst_r, zeros_hbm)


def _update2_body(h_ref, agg_ref, wa_ref, wb_ref, b1_ref, w2_ref, b2_ref, o_ref):
    h = h_ref[...]
    agg = agg_ref[0] + agg_ref[1]
    u = jax.nn.silu(
        jnp.dot(h, wa_ref[...], preferred_element_type=jnp.float32)
        + jnp.dot(agg, wb_ref[...], preferred_element_type=jnp.float32)
        + b1_ref[...])
    o_ref[...] = h + jnp.dot(u, w2_ref[...], preferred_element_type=jnp.float32) + b2_ref[...]


def _update2(h, agg2, p, block=1024):
    n = h.shape[0]
    return pl.pallas_call(
        _update2_body,
        grid=(pl.cdiv(n, block),),
        in_specs=[
            pl.BlockSpec((block, HID), lambda i: (i, 0)),
            pl.BlockSpec((2, block, HID), lambda i: (0, i, 0)),
            pl.BlockSpec((HID, HID), lambda i: (0, 0)),
            pl.BlockSpec((HID, HID), lambda i: (0, 0)),
            pl.BlockSpec((1, HID), lambda i: (0, 0)),
            pl.BlockSpec((HID, HID), lambda i: (0, 0)),
            pl.BlockSpec((1, HID), lambda i: (0, 0)),
        ],
        out_specs=pl.BlockSpec((block, HID), lambda i: (i, 0)),
        out_shape=jax.ShapeDtypeStruct((n, HID), jnp.float32),
    )(h, agg2, p['Wu1'][:HID], p['Wu1'][HID:],
      p['bu1'].reshape(1, -1), p['Wu2'], p['bu2'].reshape(1, -1))


def _prep_edges(edge_index, n_dst, blk=EBLK):
    src = edge_index[0]
    dst = edge_index[1]
    e_num = src.shape[0]
    e_pad = _pad_to(e_num, blk)
    src_r = jnp.pad(src, (0, e_pad - e_num)).astype(jnp.int32).reshape(e_pad // 128, 128)
    dst_r = jnp.pad(dst, (0, e_pad - e_num),
                    constant_values=n_dst).astype(jnp.int32).reshape(e_pad // 128, 128)
    return src_r, dst_r, e_pad


def _round_sc(p, h_src, h_dst, src_r, dst_r, e, zeros_hbm, n_pad, nblk):
    hm = _affine_silu(h_src, p['Wm'], p['bm'])
    agg = _edge_sc(hm, e, src_r, dst_r, zeros_hbm, n_pad, nblk)
    return _update(h_dst, agg, p)


def _round_jnp(p, h_src, h_dst, src, dst, e, n_dst):
    hm = jax.nn.silu(h_src @ p['Wm'] + p['bm'])
    m = jnp.take(hm, src, axis=0) * e
    agg = jax.ops.segment_sum(m, dst, num_segments=n_dst)
    u = jnp.concatenate([h_dst, agg], axis=-1)
    return h_dst + jax.nn.silu(u @ p['Wu1'] + p['bu1']) @ p['Wu2'] + p['bu2']


def kernel(z, bond_edge_index, bond_edge_attr, aq_edge_index, aq_edge_attr,
           qq_edge_index, qq_edge_attr, n_query, params):
    p = params
    zeros_hbm = jnp.zeros((128, HID), jnp.float32)
    h_atom = jnp.take(p['emb'], z, axis=0)

    n_pad_b = _pad_to(N_ATOM + 1, 1024)
    src_b, dst_b, e_pad_b = _prep_edges(bond_edge_index, N_ATOM, blk=32 * C_EDGE)
    attr_b = jnp.pad(bond_edge_attr, ((0, e_pad_b - bond_edge_attr.shape[0]), (0, 0)))
    e_b = _affine_silu(attr_b, p['bond']['Wr'], p['bond']['br'])
    nblk32_b = e_pad_b // (32 * C_EDGE)
    for _ in range(2):
        hm_b = _affine_silu(h_atom, p['bond']['Wm'], p['bond']['bm'])
        agg2_b = _edge_sc_partial(hm_b, e_b, src_b, dst_b, zeros_hbm,
                                  n_pad_b, nblk32_b)
        h_atom = _update2(h_atom, agg2_b, p['bond'])

    n_pad_q = _pad_to(N_QUERY + 1, 1024)
    src_a, dst_a, e_pad_a = _prep_edges(aq_edge_index, N_QUERY)
    attr_a = jnp.pad(aq_edge_attr, ((0, e_pad_a - aq_edge_attr.shape[0]), (0, 0)))
    e_a = _affine_silu(attr_a, p['aq']['Wr'], p['aq']['br'])
    nblk_a = e_pad_a // EBLK
    h_query = jnp.zeros((N_QUERY, HID), jnp.float32)
    for _ in range(3):
        h_query = _round_sc(p['aq'], h_atom, h_query, src_a, dst_a, e_a,
                            zeros_hbm, n_pad_q, nblk_a)

    src_q, dst_q, e_pad_q = _prep_edges(qq_edge_index, N_QUERY)
    attr_q = jnp.pad(qq_edge_attr, ((0, e_pad_q - qq_edge_attr.shape[0]), (0, 0)))
    e_q = _affine_silu(attr_q, p['qq']['Wr'], p['qq']['br'])
    nblk_q = e_pad_q // EBLK
    for _ in range(2):
        h_query = _round_sc(p['qq'], h_query, h_query, src_q, dst_q, e_q,
                            zeros_hbm, n_pad_q, nblk_q)

    return _head(h_query, p).reshape(N_QUERY)
